# baseline (speedup 1.0000x reference)
import functools

import jax
import jax.numpy as jnp
from jax import lax
from jax.experimental import pallas as pl
from jax.experimental.pallas import tpu as pltpu
from jax.experimental.pallas import tpu_sc as plsc

DIM = 128
HALF = 64
L_SIZE = 10000
C_SIZE = 5000
N_EDGES = 320000
N_ITER = 4

L_PAD = 10240
C_PAD = 5120
NC = 2
NS = 16
K = 256
E_PAD = 327680
EPS = E_PAD // NS
CHUNKS = EPS // K
SUPER = 2
NSUP = CHUNKS // SUPER

_f32 = jnp.float32


def _dot(x, w):
    return lax.dot_general(x, w, (((1,), (1,)), ((), ())),
                           preferred_element_type=_f32)


def _pre_body(x_ref, w1_ref, b1_ref, w2_ref, b2_ref, msg_ref):
    x = x_ref[...]
    h = jnp.maximum(_dot(x, w1_ref[...]) + b1_ref[...], 0.0)
    y = _dot(h, w2_ref[...]) + b2_ref[...]
    msg_ref[0] = y[:, :HALF]
    msg_ref[1] = y[:, HALF:]


def _gru_gates(gi, gh, h):
    ir, iz, inn = gi[:, :DIM], gi[:, DIM:2 * DIM], gi[:, 2 * DIM:]
    hr, hz, hn = gh[:, :DIM], gh[:, DIM:2 * DIM], gh[:, 2 * DIM:]
    r = jax.nn.sigmoid(ir + hr)
    z = jax.nn.sigmoid(iz + hz)
    n = jnp.tanh(inn + r * hn)
    return (1.0 - z) * n + z * h


def _gru_c_body(agg_ref, h_ref, wih_ref, whh_ref, bih_ref, bhh_ref, out_ref):
    x = jnp.concatenate([agg_ref[0], agg_ref[1]], axis=1)
    h = h_ref[...]
    gi = _dot(x, wih_ref[...]) + bih_ref[...]
    gh = _dot(h, whh_ref[...]) + bhh_ref[...]
    out_ref[...] = _gru_gates(gi, gh, h)


def _gru_l_body(agg_ref, h_ref, v1_ref, vb1_ref, v2_ref, vb2_ref,
                wih_ref, whh_ref, bih_ref, bhh_ref, out_ref):
    h = h_ref[...]
    xs = h.reshape(-1, 2, DIM)
    xsw = jnp.concatenate([xs[:, 1:2, :], xs[:, 0:1, :]], axis=1)
    xsw = xsw.reshape(h.shape)
    h2 = jnp.maximum(_dot(xsw, v1_ref[...]) + vb1_ref[...], 0.0)
    l2l = _dot(h2, v2_ref[...]) + vb2_ref[...]
    x = jnp.concatenate([agg_ref[0], agg_ref[1], l2l], axis=1)
    gi = _dot(x, wih_ref[...]) + bih_ref[...]
    gh = _dot(h, whh_ref[...]) + bhh_ref[...]
    out_ref[...] = _gru_gates(gi, gh, h)


def _full(shape):
    return pl.BlockSpec(shape, lambda i: tuple(0 for _ in shape))


def _rows(block, width=DIM):
    return pl.BlockSpec((block, width), lambda i: (i, 0))


def _agg_spec(blk):
    return pl.BlockSpec((2, blk, HALF), lambda i: (0, i, 0))


_BLK = 1024


def _pre(x, w1, b1, w2, b2, n_pad, blk):
    return pl.pallas_call(
        _pre_body,
        grid=(n_pad // blk,),
        in_specs=[_rows(blk), _full((DIM, DIM)), _full((1, DIM)),
                  _full((DIM, DIM)), _full((1, DIM))],
        out_specs=_agg_spec(blk),
        out_shape=jax.ShapeDtypeStruct((2, n_pad, HALF), _f32),
    )(x, w1, b1.reshape(1, DIM), w2, b2.reshape(1, DIM))


def _gru_c(agg, h, wih, whh, bih, bhh):
    blk = 5120
    return pl.pallas_call(
        _gru_c_body,
        grid=(C_PAD // blk,),
        in_specs=[_agg_spec(blk), _rows(blk), _full((3 * DIM, DIM)),
                  _full((3 * DIM, DIM)), _full((1, 3 * DIM)),
                  _full((1, 3 * DIM))],
        out_specs=_rows(blk),
        out_shape=jax.ShapeDtypeStruct((C_PAD, DIM), _f32),
    )(agg, h, wih, whh, bih.reshape(1, 3 * DIM), bhh.reshape(1, 3 * DIM))


def _gru_l(agg, h, v1, vb1, v2, vb2, wih, whh, bih, bhh):
    blk = 5120
    return pl.pallas_call(
        _gru_l_body,
        grid=(L_PAD // blk,),
        in_specs=[_agg_spec(blk), _rows(blk),
                  _full((DIM, DIM)), _full((1, DIM)),
                  _full((DIM, DIM)), _full((1, DIM)),
                  _full((3 * DIM, 2 * DIM)), _full((3 * DIM, DIM)),
                  _full((1, 3 * DIM)), _full((1, 3 * DIM))],
        out_specs=_rows(blk),
        out_shape=jax.ShapeDtypeStruct((L_PAD, DIM), _f32),
    )(agg, h, v1, vb1.reshape(1, DIM), v2, vb2.reshape(1, DIM),
      wih, whh, bih.reshape(1, 3 * DIM), bhh.reshape(1, 3 * DIM))


_ZR = 64
_C_STRIPE = C_PAD // NS
_L_STRIPE = L_PAD // NS


def _sc_body(l_idx, c_idx, l_tab, c_tab, out_c, out_l,
             lidx_blk, cidx_blk, lrows_v, crows_v, cacc, lacc,
             semg_a, semg_b, sems_a, sems_b, sem_i):
    cid = lax.axis_index("c")
    sid = lax.axis_index("s")

    zeros16 = jnp.zeros((16,), _f32)

    def _zrow(i, carry):
        for j in range(HALF // 16):
            lrows_v[0, i, pl.ds(j * 16, 16)] = zeros16
        return carry

    lax.fori_loop(0, _ZR, _zrow, 0)
    zsrc = lrows_v.at[0, pl.ds(0, _ZR)]

    def _zc(j, carry):
        base = pl.multiple_of(sid * _C_STRIPE + j * _ZR, _ZR)
        pltpu.sync_copy(zsrc, cacc.at[pl.ds(base, _ZR)])
        return carry

    lax.fori_loop(0, _C_STRIPE // _ZR, _zc, 0)

    def _zl(j, carry):
        base = pl.multiple_of(sid * _L_STRIPE + j * _ZR, _ZR)
        pltpu.sync_copy(zsrc, lacc.at[pl.ds(base, _ZR)])
        return carry

    lax.fori_loop(0, _L_STRIPE // _ZR, _zl, 0)
    plsc.subcore_barrier()

    def _idx_src(sup):
        base = pl.multiple_of(sid * CHUNKS + sup * SUPER, 8)
        return (l_idx.at[pl.ds(base, SUPER)], c_idx.at[pl.ds(base, SUPER)])

    def _fire_g(qq, u, buf, sem):
        pltpu.async_copy(l_tab.at[cid].at[lidx_blk.at[qq, u]],
                         lrows_v.at[buf], sem)
        pltpu.async_copy(c_tab.at[cid].at[cidx_blk.at[qq, u]],
                         crows_v.at[buf], sem)

    def _wait_g(qq, u, buf, sem):
        pltpu.make_async_copy(l_tab.at[cid].at[lidx_blk.at[qq, u]],
                              lrows_v.at[buf], sem).wait()
        pltpu.make_async_copy(c_tab.at[cid].at[cidx_blk.at[qq, u]],
                              crows_v.at[buf], sem).wait()

    def _fire_s(qq, u, buf, sem):
        pltpu.async_copy(lrows_v.at[buf], cacc.at[cidx_blk.at[qq, u]],
                         sem, add=True)
        pltpu.async_copy(crows_v.at[buf], lacc.at[lidx_blk.at[qq, u]],
                         sem, add=True)

    def _wait_s(qq, u, buf, sem):
        pltpu.make_async_copy(lrows_v.at[buf],
                              cacc.at[cidx_blk.at[qq, u]], sem).wait()
        pltpu.make_async_copy(crows_v.at[buf],
                              lacc.at[lidx_blk.at[qq, u]], sem).wait()

    lsrc0, csrc0 = _idx_src(0)
    pltpu.sync_copy(lsrc0, lidx_blk.at[0])
    pltpu.sync_copy(csrc0, cidx_blk.at[0])
    _fire_g(0, 0, 0, semg_a)

    def _sup_body(s, carry):
        q = lax.rem(s, 2)
        nq = 1 - q

        @pl.when(s > 0)
        def _():
            _wait_s(nq, 1, 1, sems_b)

        @pl.when(s < NSUP - 1)
        def _():
            lsrc, csrc = _idx_src(s + 1)
            pltpu.async_copy(lsrc, lidx_blk.at[nq], sem_i)
            pltpu.async_copy(csrc, cidx_blk.at[nq], sem_i)

        _fire_g(q, 1, 1, semg_b)
        _wait_g(q, 0, 0, semg_a)
        _fire_s(q, 0, 0, sems_a)

        _wait_s(q, 0, 0, sems_a)

        @pl.when(s < NSUP - 1)
        def _():
            lsrc, csrc = _idx_src(s + 1)
            pltpu.make_async_copy(lsrc, lidx_blk.at[nq], sem_i).wait()
            pltpu.make_async_copy(csrc, cidx_blk.at[nq], sem_i).wait()
            _fire_g(nq, 0, 0, semg_a)
        _wait_g(q, 1, 1, semg_b)
        _fire_s(q, 1, 1, sems_b)
        return carry

    lax.fori_loop(0, NSUP, _sup_body, 0)
    _wait_s(lax.rem(NSUP - 1, 2), 1, 1, sems_b)
    plsc.subcore_barrier()

    cbase = pl.multiple_of(sid * _C_STRIPE, _C_STRIPE)
    pltpu.sync_copy(cacc.at[pl.ds(cbase, _C_STRIPE)],
                    out_c.at[cid, pl.ds(cbase, _C_STRIPE)])
    lbase = pl.multiple_of(sid * _L_STRIPE, _L_STRIPE)
    pltpu.sync_copy(lacc.at[pl.ds(lbase, _L_STRIPE)],
                    out_l.at[cid, pl.ds(lbase, _L_STRIPE)])


_sc_agg = functools.partial(
    pl.kernel,
    out_type=(jax.ShapeDtypeStruct((NC, C_PAD, HALF), _f32),
              jax.ShapeDtypeStruct((NC, L_PAD, HALF), _f32)),
    mesh=plsc.VectorSubcoreMesh(core_axis_name="c", subcore_axis_name="s"),
    compiler_params=pltpu.CompilerParams(use_tc_tiling_on_sc=False),
    scratch_types=[
        pltpu.VMEM((2, SUPER, K), jnp.int32),
        pltpu.VMEM((2, SUPER, K), jnp.int32),
        pltpu.VMEM((2, K, HALF), _f32),
        pltpu.VMEM((2, K, HALF), _f32),
        pltpu.VMEM_SHARED((C_PAD, HALF), _f32),
        pltpu.VMEM_SHARED((L_PAD, HALF), _f32),
        pltpu.SemaphoreType.DMA,
        pltpu.SemaphoreType.DMA,
        pltpu.SemaphoreType.DMA,
        pltpu.SemaphoreType.DMA,
        pltpu.SemaphoreType.DMA,
    ],
)(_sc_body)


def kernel(l_size, c_size, l_edge_index, c_edge_index, l_emb, c_emb,
           l2c_W1, l2c_b1, l2c_W2, l2c_b2, c2l_W1, c2l_b1, c2l_W2, c2l_b2,
           l2l_W1, l2l_b1, l2l_W2, l2l_b2, cu_Wih, cu_Whh, cu_bih, cu_bhh,
           lu_Wih, lu_Whh, lu_bih, lu_bhh):
    pad_e = E_PAD - N_EDGES
    l_idx = jnp.concatenate(
        [l_edge_index.astype(jnp.int32),
         jnp.full((pad_e,), L_SIZE, jnp.int32)])
    c_idx = jnp.concatenate(
        [c_edge_index.astype(jnp.int32),
         jnp.full((pad_e,), C_SIZE, jnp.int32)])

    l_emb_p = jnp.pad(l_emb, ((0, L_PAD - L_SIZE), (0, 0)))
    c_emb_p = jnp.pad(c_emb, ((0, C_PAD - C_SIZE), (0, 0)))

    l_embs = [l_emb]
    c_embs = [c_emb]
    l_idx2 = l_idx.reshape(E_PAD // K, K)
    c_idx2 = c_idx.reshape(E_PAD // K, K)
    for _ in range(N_ITER):
        l_msg = _pre(l_emb_p, l2c_W1, l2c_b1, l2c_W2, l2c_b2, L_PAD, 10240)
        c_msg = _pre(c_emb_p, c2l_W1, c2l_b1, c2l_W2, c2l_b2, C_PAD, 5120)
        agg_c, agg_l = _sc_agg(l_idx2, c_idx2, l_msg, c_msg)
        c_emb_p = _gru_c(agg_c, c_emb_p, cu_Wih, cu_Whh, cu_bih, cu_bhh)
        l_emb_p = _gru_l(agg_l, l_emb_p, l2l_W1, l2l_b1, l2l_W2, l2l_b2,
                         lu_Wih, lu_Whh, lu_bih, lu_bhh)
        l_embs.append(l_emb_p[:L_SIZE])
        c_embs.append(c_emb_p[:C_SIZE])

    return (jnp.stack(l_embs), jnp.stack(c_embs))


# speedup vs baseline: 1.0173x; 1.0173x over previous
import functools

import jax
import jax.numpy as jnp
from jax import lax
from jax.experimental import pallas as pl
from jax.experimental.pallas import tpu as pltpu
from jax.experimental.pallas import tpu_sc as plsc

DIM = 128
HALF = 64
L_SIZE = 10000
C_SIZE = 5000
N_EDGES = 320000
N_ITER = 4

L_PAD = 10240
C_PAD = 5120
NC = 2
NS = 16
K = 256
E_PAD = 327680
EPS = E_PAD // NS
CHUNKS = EPS // K
SUPER = 2
NSUP = CHUNKS // SUPER

_f32 = jnp.float32


def _dot(x, w):
    return lax.dot_general(x, w, (((1,), (1,)), ((), ())),
                           preferred_element_type=_f32)


def _pre_body(x_ref, w1_ref, b1_ref, w2_ref, b2_ref, msg_ref):
    x = x_ref[...]
    h = jnp.maximum(_dot(x, w1_ref[...]) + b1_ref[...], 0.0)
    y = _dot(h, w2_ref[...]) + b2_ref[...]
    msg_ref[0] = y[:, :HALF]
    msg_ref[1] = y[:, HALF:]


def _gru_gates(gi, gh, h):
    ir, iz, inn = gi[:, :DIM], gi[:, DIM:2 * DIM], gi[:, 2 * DIM:]
    hr, hz, hn = gh[:, :DIM], gh[:, DIM:2 * DIM], gh[:, 2 * DIM:]
    r = jax.nn.sigmoid(ir + hr)
    z = jax.nn.sigmoid(iz + hz)
    n = jnp.tanh(inn + r * hn)
    return (1.0 - z) * n + z * h


def _gru_c_body(agg_ref, h_ref, wih_ref, whh_ref, bih_ref, bhh_ref, out_ref):
    x = jnp.concatenate([agg_ref[0], agg_ref[1]], axis=1)
    h = h_ref[...]
    gi = _dot(x, wih_ref[...]) + bih_ref[...]
    gh = _dot(h, whh_ref[...]) + bhh_ref[...]
    out_ref[...] = _gru_gates(gi, gh, h)


def _gru_l_body(agg_ref, h_ref, v1_ref, vb1_ref, v2_ref, vb2_ref,
                wih_ref, whh_ref, bih_ref, bhh_ref, out_ref):
    h = h_ref[...]
    xs = h.reshape(-1, 2, DIM)
    xsw = jnp.concatenate([xs[:, 1:2, :], xs[:, 0:1, :]], axis=1)
    xsw = xsw.reshape(h.shape)
    h2 = jnp.maximum(_dot(xsw, v1_ref[...]) + vb1_ref[...], 0.0)
    l2l = _dot(h2, v2_ref[...]) + vb2_ref[...]
    x = jnp.concatenate([agg_ref[0], agg_ref[1], l2l], axis=1)
    gi = _dot(x, wih_ref[...]) + bih_ref[...]
    gh = _dot(h, whh_ref[...]) + bhh_ref[...]
    out_ref[...] = _gru_gates(gi, gh, h)


def _full(shape):
    return pl.BlockSpec(shape, lambda i: tuple(0 for _ in shape))


def _rows(block, width=DIM):
    return pl.BlockSpec((block, width), lambda i: (i, 0))


def _agg_spec(blk):
    return pl.BlockSpec((2, blk, HALF), lambda i: (0, i, 0))


_BLK = 1024


def _pre(x, w1, b1, w2, b2, n_pad, blk):
    return pl.pallas_call(
        _pre_body,
        grid=(n_pad // blk,),
        in_specs=[_rows(blk), _full((DIM, DIM)), _full((1, DIM)),
                  _full((DIM, DIM)), _full((1, DIM))],
        out_specs=_agg_spec(blk),
        out_shape=jax.ShapeDtypeStruct((2, n_pad, HALF), _f32),
    )(x, w1, b1.reshape(1, DIM), w2, b2.reshape(1, DIM))


def _gru_c(agg, h, wih, whh, bih, bhh):
    blk = 2560
    return pl.pallas_call(
        _gru_c_body,
        grid=(C_PAD // blk,),
        in_specs=[_agg_spec(blk), _rows(blk), _full((3 * DIM, DIM)),
                  _full((3 * DIM, DIM)), _full((1, 3 * DIM)),
                  _full((1, 3 * DIM))],
        out_specs=_rows(blk),
        out_shape=jax.ShapeDtypeStruct((C_PAD, DIM), _f32),
    )(agg, h, wih, whh, bih.reshape(1, 3 * DIM), bhh.reshape(1, 3 * DIM))


def _gru_l(agg, h, v1, vb1, v2, vb2, wih, whh, bih, bhh):
    blk = 2560
    return pl.pallas_call(
        _gru_l_body,
        grid=(L_PAD // blk,),
        in_specs=[_agg_spec(blk), _rows(blk),
                  _full((DIM, DIM)), _full((1, DIM)),
                  _full((DIM, DIM)), _full((1, DIM)),
                  _full((3 * DIM, 2 * DIM)), _full((3 * DIM, DIM)),
                  _full((1, 3 * DIM)), _full((1, 3 * DIM))],
        out_specs=_rows(blk),
        out_shape=jax.ShapeDtypeStruct((L_PAD, DIM), _f32),
    )(agg, h, v1, vb1.reshape(1, DIM), v2, vb2.reshape(1, DIM),
      wih, whh, bih.reshape(1, 3 * DIM), bhh.reshape(1, 3 * DIM))


_ZR = 64
_C_STRIPE = C_PAD // NS
_L_STRIPE = L_PAD // NS


def _sc_body(l_idx, c_idx, l_tab, c_tab, out_c, out_l,
             lidx_blk, cidx_blk, lrows_v, crows_v, cacc, lacc,
             semg_a, semg_b, sems_a, sems_b, sem_i):
    cid = lax.axis_index("c")
    sid = lax.axis_index("s")

    zeros16 = jnp.zeros((16,), _f32)

    def _zrow(i, carry):
        for j in range(HALF // 16):
            lrows_v[0, i, pl.ds(j * 16, 16)] = zeros16
        return carry

    lax.fori_loop(0, _ZR, _zrow, 0)
    zsrc = lrows_v.at[0, pl.ds(0, _ZR)]

    def _zc(j, carry):
        base = pl.multiple_of(sid * _C_STRIPE + j * _ZR, _ZR)
        pltpu.sync_copy(zsrc, cacc.at[pl.ds(base, _ZR)])
        return carry

    lax.fori_loop(0, _C_STRIPE // _ZR, _zc, 0)

    def _zl(j, carry):
        base = pl.multiple_of(sid * _L_STRIPE + j * _ZR, _ZR)
        pltpu.sync_copy(zsrc, lacc.at[pl.ds(base, _ZR)])
        return carry

    lax.fori_loop(0, _L_STRIPE // _ZR, _zl, 0)
    plsc.subcore_barrier()

    def _idx_src(sup):
        base = pl.multiple_of(sid * CHUNKS + sup * SUPER, 8)
        return (l_idx.at[pl.ds(base, SUPER)], c_idx.at[pl.ds(base, SUPER)])

    def _fire_g(qq, u, buf, sem):
        pltpu.async_copy(l_tab.at[cid].at[lidx_blk.at[qq, u]],
                         lrows_v.at[buf], sem)
        pltpu.async_copy(c_tab.at[cid].at[cidx_blk.at[qq, u]],
                         crows_v.at[buf], sem)

    def _wait_g(qq, u, buf, sem):
        pltpu.make_async_copy(l_tab.at[cid].at[lidx_blk.at[qq, u]],
                              lrows_v.at[buf], sem).wait()
        pltpu.make_async_copy(c_tab.at[cid].at[cidx_blk.at[qq, u]],
                              crows_v.at[buf], sem).wait()

    def _fire_s(qq, u, buf, sem):
        pltpu.async_copy(lrows_v.at[buf], cacc.at[cidx_blk.at[qq, u]],
                         sem, add=True)
        pltpu.async_copy(crows_v.at[buf], lacc.at[lidx_blk.at[qq, u]],
                         sem, add=True)

    def _wait_s(qq, u, buf, sem):
        pltpu.make_async_copy(lrows_v.at[buf],
                              cacc.at[cidx_blk.at[qq, u]], sem).wait()
        pltpu.make_async_copy(crows_v.at[buf],
                              lacc.at[lidx_blk.at[qq, u]], sem).wait()

    lsrc0, csrc0 = _idx_src(0)
    pltpu.sync_copy(lsrc0, lidx_blk.at[0])
    pltpu.sync_copy(csrc0, cidx_blk.at[0])
    _fire_g(0, 0, 0, semg_a)

    def _sup_body(s, carry):
        q = lax.rem(s, 2)
        nq = 1 - q

        @pl.when(s > 0)
        def _():
            _wait_s(nq, 1, 1, sems_b)

        @pl.when(s < NSUP - 1)
        def _():
            lsrc, csrc = _idx_src(s + 1)
            pltpu.async_copy(lsrc, lidx_blk.at[nq], sem_i)
            pltpu.async_copy(csrc, cidx_blk.at[nq], sem_i)

        _fire_g(q, 1, 1, semg_b)
        _wait_g(q, 0, 0, semg_a)
        _fire_s(q, 0, 0, sems_a)

        _wait_s(q, 0, 0, sems_a)

        @pl.when(s < NSUP - 1)
        def _():
            lsrc, csrc = _idx_src(s + 1)
            pltpu.make_async_copy(lsrc, lidx_blk.at[nq], sem_i).wait()
            pltpu.make_async_copy(csrc, cidx_blk.at[nq], sem_i).wait()
            _fire_g(nq, 0, 0, semg_a)
        _wait_g(q, 1, 1, semg_b)
        _fire_s(q, 1, 1, sems_b)
        return carry

    lax.fori_loop(0, NSUP, _sup_body, 0)
    _wait_s(lax.rem(NSUP - 1, 2), 1, 1, sems_b)
    plsc.subcore_barrier()

    cbase = pl.multiple_of(sid * _C_STRIPE, _C_STRIPE)
    pltpu.sync_copy(cacc.at[pl.ds(cbase, _C_STRIPE)],
                    out_c.at[cid, pl.ds(cbase, _C_STRIPE)])
    lbase = pl.multiple_of(sid * _L_STRIPE, _L_STRIPE)
    pltpu.sync_copy(lacc.at[pl.ds(lbase, _L_STRIPE)],
                    out_l.at[cid, pl.ds(lbase, _L_STRIPE)])


_sc_agg = functools.partial(
    pl.kernel,
    out_type=(jax.ShapeDtypeStruct((NC, C_PAD, HALF), _f32),
              jax.ShapeDtypeStruct((NC, L_PAD, HALF), _f32)),
    mesh=plsc.VectorSubcoreMesh(core_axis_name="c", subcore_axis_name="s"),
    compiler_params=pltpu.CompilerParams(use_tc_tiling_on_sc=False),
    scratch_types=[
        pltpu.VMEM((2, SUPER, K), jnp.int32),
        pltpu.VMEM((2, SUPER, K), jnp.int32),
        pltpu.VMEM((2, K, HALF), _f32),
        pltpu.VMEM((2, K, HALF), _f32),
        pltpu.VMEM_SHARED((C_PAD, HALF), _f32),
        pltpu.VMEM_SHARED((L_PAD, HALF), _f32),
        pltpu.SemaphoreType.DMA,
        pltpu.SemaphoreType.DMA,
        pltpu.SemaphoreType.DMA,
        pltpu.SemaphoreType.DMA,
        pltpu.SemaphoreType.DMA,
    ],
)(_sc_body)


def kernel(l_size, c_size, l_edge_index, c_edge_index, l_emb, c_emb,
           l2c_W1, l2c_b1, l2c_W2, l2c_b2, c2l_W1, c2l_b1, c2l_W2, c2l_b2,
           l2l_W1, l2l_b1, l2l_W2, l2l_b2, cu_Wih, cu_Whh, cu_bih, cu_bhh,
           lu_Wih, lu_Whh, lu_bih, lu_bhh):
    pad_e = E_PAD - N_EDGES
    l_idx = jnp.concatenate(
        [l_edge_index.astype(jnp.int32),
         jnp.full((pad_e,), L_SIZE, jnp.int32)])
    c_idx = jnp.concatenate(
        [c_edge_index.astype(jnp.int32),
         jnp.full((pad_e,), C_SIZE, jnp.int32)])

    l_emb_p = jnp.pad(l_emb, ((0, L_PAD - L_SIZE), (0, 0)))
    c_emb_p = jnp.pad(c_emb, ((0, C_PAD - C_SIZE), (0, 0)))

    l_embs = [l_emb]
    c_embs = [c_emb]
    l_idx2 = l_idx.reshape(E_PAD // K, K)
    c_idx2 = c_idx.reshape(E_PAD // K, K)
    for _ in range(N_ITER):
        l_msg = _pre(l_emb_p, l2c_W1, l2c_b1, l2c_W2, l2c_b2, L_PAD, 2560)
        c_msg = _pre(c_emb_p, c2l_W1, c2l_b1, c2l_W2, c2l_b2, C_PAD, 2560)
        agg_c, agg_l = _sc_agg(l_idx2, c_idx2, l_msg, c_msg)
        c_emb_p = _gru_c(agg_c, c_emb_p, cu_Wih, cu_Whh, cu_bih, cu_bhh)
        l_emb_p = _gru_l(agg_l, l_emb_p, l2l_W1, l2l_b1, l2l_W2, l2l_b2,
                         lu_Wih, lu_Whh, lu_bih, lu_bhh)
        l_embs.append(l_emb_p[:L_SIZE])
        c_embs.append(c_emb_p[:C_SIZE])

    return (jnp.stack(l_embs), jnp.stack(c_embs))
